# baseline (speedup 1.0000x reference)
import functools

import jax
import jax.numpy as jnp
from jax import lax
from jax.experimental import pallas as pl
from jax.experimental.pallas import tpu as pltpu
from jax.experimental.pallas import tpu_sc as plsc

N = 10000
E = 320000
D = 128
NSUB = 16
EPC = E // 2
EPT = EPC // NSUB
CHUNK = 80
NCHUNK = -(-EPT // CHUNK)
EPTP = NCHUNK * CHUNK
NPAD = 8
WB_A = 624
WB_B = N - 15 * WB_A

_MM_BLK = 1000


def _mm1(x, w):
    def body(x_ref, w_ref, o_ref):
        o_ref[...] = lax.dot_general(
            x_ref[...], w_ref[...], (((1,), (0,)), ((), ())),
            preferred_element_type=jnp.float32)

    return pl.pallas_call(
        body,
        grid=(N // _MM_BLK,),
        in_specs=[
            pl.BlockSpec((_MM_BLK, D), lambda i: (i, 0)),
            pl.BlockSpec((D, D), lambda i: (0, 0)),
        ],
        out_specs=pl.BlockSpec((_MM_BLK, D), lambda i: (i, 0)),
        out_shape=jax.ShapeDtypeStruct((N, D), jnp.float32),
    )(x, w)


def _mm2(pp, w):
    def body(a_ref, b_ref, w_ref, o_ref):
        h = jnp.maximum(a_ref[0] + b_ref[0], 0.0)
        o_ref[...] = lax.dot_general(
            h, w_ref[...], (((1,), (0,)), ((), ())),
            preferred_element_type=jnp.float32)

    return pl.pallas_call(
        body,
        grid=(N // _MM_BLK,),
        in_specs=[
            pl.BlockSpec((1, _MM_BLK, D), lambda i: (0, i, 0)),
            pl.BlockSpec((1, _MM_BLK, D), lambda i: (1, i, 0)),
            pl.BlockSpec((D, D), lambda i: (0, 0)),
        ],
        out_specs=pl.BlockSpec((_MM_BLK, D), lambda i: (i, 0)),
        out_shape=jax.ShapeDtypeStruct((N, D), jnp.float32),
    )(pp, pp, w)


def _combine(pp):
    def body(a_ref, b_ref, o_ref):
        o_ref[...] = a_ref[0] + b_ref[0]

    return pl.pallas_call(
        body,
        grid=(N // _MM_BLK,),
        in_specs=[
            pl.BlockSpec((1, _MM_BLK, D), lambda i: (0, i, 0)),
            pl.BlockSpec((1, _MM_BLK, D), lambda i: (1, i, 0)),
        ],
        out_specs=pl.BlockSpec((_MM_BLK, D), lambda i: (i, 0)),
        out_shape=jax.ShapeDtypeStruct((N, D), jnp.float32),
    )(pp, pp)


NBUF = 4
ISLOT = NBUF + 1


def _agg(table, row3, col3, zrs):
    mesh = plsc.VectorSubcoreMesh(core_axis_name="c", subcore_axis_name="s")

    @functools.partial(
        pl.kernel,
        mesh=mesh,
        out_type=jax.ShapeDtypeStruct((2, N, D), jnp.float32),
        scratch_types=[
            pltpu.VMEM((ISLOT, CHUNK), jnp.int32),
            pltpu.VMEM((ISLOT, CHUNK), jnp.int32),
            pltpu.VMEM((NBUF, CHUNK, D), jnp.float32),
            pltpu.VMEM_SHARED((N + NPAD, D), jnp.float32),
            pltpu.SemaphoreType.DMA((ISLOT,)),
            pltpu.SemaphoreType.DMA((ISLOT,)),
            pltpu.SemaphoreType.DMA((NBUF,)),
        ],
    )
    def agg(table_ref, row_ref, col_ref, zrs_ref, out_ref,
            idxr, idxc, bufs, acc, rsem, csem, gsem):
        cid = lax.axis_index("c")
        sid = lax.axis_index("s")
        wid = cid * NSUB + sid

        def idx_load(j, s):
            pltpu.async_copy(row_ref.at[wid, j], idxr.at[s], rsem.at[s])
            pltpu.async_copy(col_ref.at[wid, j], idxc.at[s], csem.at[s])

        def idx_wait(j, s):
            pltpu.make_async_copy(
                row_ref.at[wid, j], idxr.at[s], rsem.at[s]).wait()
            pltpu.make_async_copy(
                col_ref.at[wid, j], idxc.at[s], csem.at[s]).wait()

        def gather(s, b):
            pltpu.async_copy(table_ref.at[idxc.at[s]], bufs.at[b], gsem.at[b])

        def gather_wait(s, b):
            pltpu.make_async_copy(
                table_ref.at[idxc.at[s]], bufs.at[b], gsem.at[b]).wait()

        for j in range(NBUF):
            idx_load(j, j)
        for j in range(NBUF):
            idx_wait(j, j)
            gather(j, j)

        wbase = sid * WB_A

        @pl.when(sid < NSUB - 1)
        def _():
            pltpu.sync_copy(zrs_ref.at[pl.ds(0, WB_A)],
                            acc.at[pl.ds(wbase, WB_A)])

        @pl.when(sid == NSUB - 1)
        def _():
            pltpu.sync_copy(zrs_ref, acc.at[pl.ds(15 * WB_A, WB_B)])

        plsc.subcore_barrier()

        def body(i, carry):
            b = lax.rem(i, NBUF)
            s = lax.rem(i, ISLOT)
            j = i + NBUF
            sj = lax.rem(j, ISLOT)
            refill = j < NCHUNK

            @pl.when(refill)
            def _():
                idx_load(j, sj)

            gather_wait(s, b)
            pltpu.sync_copy(bufs.at[b], acc.at[idxr.at[s]], add=True)

            @pl.when(refill)
            def _():
                idx_wait(j, sj)
                gather(sj, b)

            return carry

        lax.fori_loop(0, NCHUNK, body, 0)
        plsc.subcore_barrier()

        @pl.when(sid < NSUB - 1)
        def _():
            pltpu.sync_copy(acc.at[pl.ds(wbase, WB_A)],
                            out_ref.at[cid, pl.ds(wbase, WB_A)])

        @pl.when(sid == NSUB - 1)
        def _():
            pltpu.sync_copy(acc.at[pl.ds(15 * WB_A, WB_B)],
                            out_ref.at[cid, pl.ds(15 * WB_A, WB_B)])

    return agg(table, row3, col3, zrs)


def kernel(features, edge_index, W1, W2):
    ei = edge_index.astype(jnp.int32)
    row = ei[0]
    col = ei[1]
    zrs = jnp.zeros((WB_B, D), jnp.float32)
    pad = EPTP - EPT
    row3 = jnp.pad(row.reshape(2 * NSUB, EPT), ((0, 0), (0, pad)),
                   constant_values=N).reshape(2 * NSUB, NCHUNK, CHUNK)
    col3 = jnp.pad(col.reshape(2 * NSUB, EPT), ((0, 0), (0, pad)),
                   constant_values=0).reshape(2 * NSUB, NCHUNK, CHUNK)
    t1 = _mm1(features, W1)
    pp1 = _agg(t1, row3, col3, zrs)
    t2 = _mm2(pp1, W2)
    pp2 = _agg(t2, row3, col3, zrs)
    return _combine(pp2)


# speedup vs baseline: 1.0528x; 1.0528x over previous
import functools

import jax
import jax.numpy as jnp
from jax import lax
from jax.experimental import pallas as pl
from jax.experimental.pallas import tpu as pltpu
from jax.experimental.pallas import tpu_sc as plsc

N = 10000
E = 320000
D = 128
NSUB = 16
EPC = E // 2
EPT = EPC // NSUB
CHUNK = 80
NCHUNK = -(-EPT // CHUNK)
EPTP = NCHUNK * CHUNK
NPAD = 8
WB_A = 624
WB_B = N - 15 * WB_A

_MM_BLK = 2000


def _mm1(x, w):
    def body(x_ref, w_ref, o_ref):
        o_ref[...] = lax.dot_general(
            x_ref[...], w_ref[...], (((1,), (0,)), ((), ())),
            preferred_element_type=jnp.float32)

    return pl.pallas_call(
        body,
        grid=(N // _MM_BLK,),
        in_specs=[
            pl.BlockSpec((_MM_BLK, D), lambda i: (i, 0)),
            pl.BlockSpec((D, D), lambda i: (0, 0)),
        ],
        out_specs=pl.BlockSpec((_MM_BLK, D), lambda i: (i, 0)),
        out_shape=jax.ShapeDtypeStruct((N, D), jnp.float32),
    )(x, w)


def _mm2(pp, w):
    def body(a_ref, b_ref, w_ref, o_ref):
        h = jnp.maximum(a_ref[0] + b_ref[0], 0.0)
        o_ref[...] = lax.dot_general(
            h, w_ref[...], (((1,), (0,)), ((), ())),
            preferred_element_type=jnp.float32)

    return pl.pallas_call(
        body,
        grid=(N // _MM_BLK,),
        in_specs=[
            pl.BlockSpec((1, _MM_BLK, D), lambda i: (0, i, 0)),
            pl.BlockSpec((1, _MM_BLK, D), lambda i: (1, i, 0)),
            pl.BlockSpec((D, D), lambda i: (0, 0)),
        ],
        out_specs=pl.BlockSpec((_MM_BLK, D), lambda i: (i, 0)),
        out_shape=jax.ShapeDtypeStruct((N, D), jnp.float32),
    )(pp, pp, w)


def _combine(pp):
    def body(a_ref, b_ref, o_ref):
        o_ref[...] = a_ref[0] + b_ref[0]

    return pl.pallas_call(
        body,
        grid=(N // _MM_BLK,),
        in_specs=[
            pl.BlockSpec((1, _MM_BLK, D), lambda i: (0, i, 0)),
            pl.BlockSpec((1, _MM_BLK, D), lambda i: (1, i, 0)),
        ],
        out_specs=pl.BlockSpec((_MM_BLK, D), lambda i: (i, 0)),
        out_shape=jax.ShapeDtypeStruct((N, D), jnp.float32),
    )(pp, pp)


NBUF = 4
ISLOT = NBUF + 1


def _agg(table, row, col, zrs):
    mesh = plsc.VectorSubcoreMesh(core_axis_name="c", subcore_axis_name="s")

    @functools.partial(
        pl.kernel,
        mesh=mesh,
        out_type=jax.ShapeDtypeStruct((2, N, D), jnp.float32),
        scratch_types=[
            pltpu.VMEM((ISLOT, CHUNK), jnp.int32),
            pltpu.VMEM((ISLOT, CHUNK), jnp.int32),
            pltpu.VMEM((NBUF, CHUNK, D), jnp.float32),
            pltpu.VMEM_SHARED((N + NPAD, D), jnp.float32),
            pltpu.SemaphoreType.DMA((ISLOT,)),
            pltpu.SemaphoreType.DMA((ISLOT,)),
            pltpu.SemaphoreType.DMA((NBUF,)),
        ],
    )
    def agg(row_ref, col_ref, table_ref, zrs_ref, out_ref,
            idxr, idxc, bufs, acc, rsem, csem, gsem):
        cid = lax.axis_index("c")
        sid = lax.axis_index("s")
        wid = cid * NSUB + sid
        ebase = wid * EPT

        def idx_load(j, s):
            eb = ebase + j * CHUNK
            pltpu.async_copy(
                row_ref.at[pl.ds(eb, CHUNK)], idxr.at[s], rsem.at[s])
            pltpu.async_copy(
                col_ref.at[pl.ds(eb, CHUNK)], idxc.at[s], csem.at[s])

        def idx_wait(j, s):
            eb = ebase + j * CHUNK
            pltpu.make_async_copy(
                row_ref.at[pl.ds(eb, CHUNK)], idxr.at[s], rsem.at[s]).wait()
            pltpu.make_async_copy(
                col_ref.at[pl.ds(eb, CHUNK)], idxc.at[s], csem.at[s]).wait()

        def gather(s, b):
            pltpu.async_copy(table_ref.at[idxc.at[s]], bufs.at[b], gsem.at[b])

        def gather_wait(s, b):
            pltpu.make_async_copy(
                table_ref.at[idxc.at[s]], bufs.at[b], gsem.at[b]).wait()

        for j in range(NBUF):
            idx_load(j, j)
        for j in range(NBUF):
            idx_wait(j, j)
            gather(j, j)

        wbase = sid * WB_A

        @pl.when(sid < NSUB - 1)
        def _():
            pltpu.sync_copy(zrs_ref.at[pl.ds(0, WB_A)],
                            acc.at[pl.ds(wbase, WB_A)])

        @pl.when(sid == NSUB - 1)
        def _():
            pltpu.sync_copy(zrs_ref, acc.at[pl.ds(15 * WB_A, WB_B)])

        plsc.subcore_barrier()

        def body(i, carry):
            b = lax.rem(i, NBUF)
            s = lax.rem(i, ISLOT)
            j = i + NBUF
            sj = lax.rem(j, ISLOT)
            refill = j < NCHUNK

            @pl.when(refill)
            def _():
                idx_load(j, sj)

            gather_wait(s, b)
            pltpu.sync_copy(bufs.at[b], acc.at[idxr.at[s]], add=True)

            @pl.when(refill)
            def _():
                idx_wait(j, sj)
                gather(sj, b)

            return carry

        lax.fori_loop(0, NCHUNK, body, 0)
        plsc.subcore_barrier()

        @pl.when(sid < NSUB - 1)
        def _():
            pltpu.sync_copy(acc.at[pl.ds(wbase, WB_A)],
                            out_ref.at[cid, pl.ds(wbase, WB_A)])

        @pl.when(sid == NSUB - 1)
        def _():
            pltpu.sync_copy(acc.at[pl.ds(15 * WB_A, WB_B)],
                            out_ref.at[cid, pl.ds(15 * WB_A, WB_B)])

    return agg(row, col, table, zrs)


def kernel(features, edge_index, W1, W2):
    ei = edge_index.astype(jnp.int32)
    row = ei[0]
    col = ei[1]
    zrs = jnp.zeros((WB_B, D), jnp.float32)
    t1 = _mm1(features, W1)
    pp1 = _agg(t1, row, col, zrs)
    t2 = _mm2(pp1, W2)
    pp2 = _agg(t2, row, col, zrs)
    return _combine(pp2)


# speedup vs baseline: 1.1042x; 1.0488x over previous
import functools

import jax
import jax.numpy as jnp
from jax import lax
from jax.experimental import pallas as pl
from jax.experimental.pallas import tpu as pltpu
from jax.experimental.pallas import tpu_sc as plsc

N = 10000
E = 320000
D = 128
NSUB = 16
EPC = E // 2
EPT = EPC // NSUB
CHUNK = 80
NCHUNK = -(-EPT // CHUNK)
EPTP = NCHUNK * CHUNK
NPAD = 8
WB_A = 624
WB_B = N - 15 * WB_A

_MM_BLK = 2000


def _mm1(x, w):
    def body(x_ref, w_ref, o_ref):
        o_ref[...] = lax.dot_general(
            x_ref[...], w_ref[...], (((1,), (0,)), ((), ())),
            preferred_element_type=jnp.float32)

    return pl.pallas_call(
        body,
        grid=(N // _MM_BLK,),
        in_specs=[
            pl.BlockSpec((_MM_BLK, D), lambda i: (i, 0)),
            pl.BlockSpec((D, D), lambda i: (0, 0)),
        ],
        out_specs=pl.BlockSpec((_MM_BLK, D), lambda i: (i, 0)),
        out_shape=jax.ShapeDtypeStruct((N, D), jnp.float32),
    )(x, w)


def _mm2(pp, w):
    def body(a_ref, b_ref, w_ref, o_ref):
        h = jnp.maximum(a_ref[0] + b_ref[0], 0.0)
        o_ref[...] = lax.dot_general(
            h, w_ref[...], (((1,), (0,)), ((), ())),
            preferred_element_type=jnp.float32)

    return pl.pallas_call(
        body,
        grid=(N // _MM_BLK,),
        in_specs=[
            pl.BlockSpec((1, _MM_BLK, D), lambda i: (0, i, 0)),
            pl.BlockSpec((1, _MM_BLK, D), lambda i: (1, i, 0)),
            pl.BlockSpec((D, D), lambda i: (0, 0)),
        ],
        out_specs=pl.BlockSpec((_MM_BLK, D), lambda i: (i, 0)),
        out_shape=jax.ShapeDtypeStruct((N, D), jnp.float32),
    )(pp, pp, w)


def _combine(pp):
    def body(a_ref, b_ref, o_ref):
        o_ref[...] = a_ref[0] + b_ref[0]

    return pl.pallas_call(
        body,
        grid=(N // _MM_BLK,),
        in_specs=[
            pl.BlockSpec((1, _MM_BLK, D), lambda i: (0, i, 0)),
            pl.BlockSpec((1, _MM_BLK, D), lambda i: (1, i, 0)),
        ],
        out_specs=pl.BlockSpec((_MM_BLK, D), lambda i: (i, 0)),
        out_shape=jax.ShapeDtypeStruct((N, D), jnp.float32),
    )(pp, pp)


NBUF = 4
ISLOT = NBUF + 1


def _agg(table, ei1, zrs):
    mesh = plsc.VectorSubcoreMesh(core_axis_name="c", subcore_axis_name="s")

    @functools.partial(
        pl.kernel,
        mesh=mesh,
        out_type=jax.ShapeDtypeStruct((2, N, D), jnp.float32),
        scratch_types=[
            pltpu.VMEM((ISLOT, CHUNK), jnp.int32),
            pltpu.VMEM((ISLOT, CHUNK), jnp.int32),
            pltpu.VMEM((NBUF, CHUNK, D), jnp.float32),
            pltpu.VMEM_SHARED((N + NPAD, D), jnp.float32),
            pltpu.SemaphoreType.DMA((ISLOT,)),
            pltpu.SemaphoreType.DMA((ISLOT,)),
            pltpu.SemaphoreType.DMA((NBUF,)),
        ],
    )
    def agg(ei_ref, table_ref, zrs_ref, out_ref,
            idxr, idxc, bufs, acc, rsem, csem, gsem):
        cid = lax.axis_index("c")
        sid = lax.axis_index("s")
        wid = cid * NSUB + sid
        ebase = wid * EPT

        def idx_load(j, s):
            eb = ebase + j * CHUNK
            pltpu.async_copy(
                ei_ref.at[pl.ds(eb, CHUNK)], idxr.at[s], rsem.at[s])
            pltpu.async_copy(
                ei_ref.at[pl.ds(E + eb, CHUNK)], idxc.at[s], csem.at[s])

        def idx_wait(j, s):
            eb = ebase + j * CHUNK
            pltpu.make_async_copy(
                ei_ref.at[pl.ds(eb, CHUNK)], idxr.at[s], rsem.at[s]).wait()
            pltpu.make_async_copy(
                ei_ref.at[pl.ds(E + eb, CHUNK)], idxc.at[s], csem.at[s]).wait()

        def gather(s, b):
            pltpu.async_copy(table_ref.at[idxc.at[s]], bufs.at[b], gsem.at[b])

        def gather_wait(s, b):
            pltpu.make_async_copy(
                table_ref.at[idxc.at[s]], bufs.at[b], gsem.at[b]).wait()

        for j in range(NBUF):
            idx_load(j, j)
        for j in range(NBUF):
            idx_wait(j, j)
            gather(j, j)

        wbase = sid * WB_A

        @pl.when(sid < NSUB - 1)
        def _():
            pltpu.sync_copy(zrs_ref.at[pl.ds(0, WB_A)],
                            acc.at[pl.ds(wbase, WB_A)])

        @pl.when(sid == NSUB - 1)
        def _():
            pltpu.sync_copy(zrs_ref, acc.at[pl.ds(15 * WB_A, WB_B)])

        plsc.subcore_barrier()

        def body(i, carry):
            b = lax.rem(i, NBUF)
            s = lax.rem(i, ISLOT)
            j = i + NBUF
            sj = lax.rem(j, ISLOT)
            refill = j < NCHUNK

            @pl.when(refill)
            def _():
                idx_load(j, sj)

            gather_wait(s, b)
            pltpu.sync_copy(bufs.at[b], acc.at[idxr.at[s]], add=True)

            @pl.when(refill)
            def _():
                idx_wait(j, sj)
                gather(sj, b)

            return carry

        lax.fori_loop(0, NCHUNK, body, 0)
        plsc.subcore_barrier()

        @pl.when(sid < NSUB - 1)
        def _():
            pltpu.sync_copy(acc.at[pl.ds(wbase, WB_A)],
                            out_ref.at[cid, pl.ds(wbase, WB_A)])

        @pl.when(sid == NSUB - 1)
        def _():
            pltpu.sync_copy(acc.at[pl.ds(15 * WB_A, WB_B)],
                            out_ref.at[cid, pl.ds(15 * WB_A, WB_B)])

    return agg(ei1, table, zrs)


def kernel(features, edge_index, W1, W2):
    ei1 = edge_index.astype(jnp.int32).reshape(2 * E)
    zrs = jnp.zeros((WB_B, D), jnp.float32)
    t1 = _mm1(features, W1)
    pp1 = _agg(t1, ei1, zrs)
    t2 = _mm2(pp1, W2)
    pp2 = _agg(t2, ei1, zrs)
    return _combine(pp2)
